# initial kernel scaffold (unmeasured)
import jax
import jax.numpy as jnp
from jax import lax
from jax.experimental import pallas as pl
from jax.experimental.pallas import tpu as pltpu

N_DEV = 16
M = 64
D = 1024
H = 2048
RB = 4


def kernel(x, Win0, Wout0, Win1, Wout1, Win2, Wout2):
    xb = x.astype(jnp.bfloat16)
    win0, wout0, win1, wout1, win2, wout2 = (
        w.astype(jnp.bfloat16) for w in (Win0, Wout0, Win1, Wout1, Win2, Wout2)
    )

    def body(x_ref, win0_r, wout0_r, win1_r, wout1_r, win2_r, wout2_r, out_ref,
             xfull, pacc, rs_recv, rs_send,
             ag_send_sems, ag_recv_sems, rs_send_sems, rs_recv_sems):
        me = lax.axis_index("i")
        right = lax.rem(me + 1, N_DEV)
        left = lax.rem(me + N_DEV - 1, N_DEV)

        barrier = pltpu.get_barrier_semaphore()
        for nbr in (left, right):
            pl.semaphore_signal(barrier, inc=1, device_id=(nbr,),
                                device_id_type=pl.DeviceIdType.MESH)
        pl.semaphore_wait(barrier, 2)

        layers = ((win0_r, wout0_r), (win1_r, wout1_r), (win2_r, wout2_r))
        for l, (win, wout) in enumerate(layers):
            if l == 0:
                xfull[pl.ds(me, 1)] = x_ref[...][None]

            for h in range(1, N_DEV):
                sb = lax.rem(me - h + 1 + N_DEV, N_DEV)
                rb = lax.rem(me - h + N_DEV, N_DEV)
                snd = pltpu.make_async_remote_copy(
                    src_ref=xfull.at[sb], dst_ref=xfull.at[sb],
                    send_sem=ag_send_sems.at[h - 1],
                    recv_sem=ag_recv_sems.at[h - 1],
                    device_id=(right,), device_id_type=pl.DeviceIdType.MESH,
                )
                snd.start()
                snd.wait_send()
                rcv = pltpu.make_async_remote_copy(
                    src_ref=xfull.at[rb], dst_ref=xfull.at[rb],
                    send_sem=ag_send_sems.at[h - 1],
                    recv_sem=ag_recv_sems.at[h - 1],
                    device_id=(left,), device_id_type=pl.DeviceIdType.MESH,
                )
                rcv.wait_recv()

            for g in range(N_DEV // RB):
                xg = xfull[pl.ds(g * RB, RB)].reshape(RB * M, D)
                hg = jnp.maximum(
                    jnp.dot(xg, win[...], preferred_element_type=jnp.float32),
                    0.0,
                ).astype(jnp.bfloat16)
                pg = jnp.dot(hg, wout[...], preferred_element_type=jnp.float32)
                pacc[pl.ds(g * RB, RB)] = pg.reshape(RB, M, D)

            for s in range(1, N_DEV):
                cs = lax.rem(me - s + 1 + N_DEV, N_DEV)
                chunk = pacc[pl.ds(cs, 1)][0]
                if s > 1:
                    chunk = chunk + rs_recv[s - 2]
                rs_send[s % 2] = chunk
                hop = pltpu.make_async_remote_copy(
                    src_ref=rs_send.at[s % 2], dst_ref=rs_recv.at[s - 1],
                    send_sem=rs_send_sems.at[s - 1],
                    recv_sem=rs_recv_sems.at[s - 1],
                    device_id=(right,), device_id_type=pl.DeviceIdType.MESH,
                )
                hop.start()
                hop.wait()

            y = pacc[pl.ds(me, 1)][0] + rs_recv[N_DEV - 2]
            if l < 2:
                xfull[pl.ds(me, 1)] = y.astype(jnp.bfloat16)[None]
            else:
                out_ref[...] = y

    return pl.pallas_call(
        body,
        out_shape=jax.ShapeDtypeStruct((M, D), jnp.float32),
        in_specs=[pl.BlockSpec(memory_space=pltpu.VMEM)] * 7,
        out_specs=pl.BlockSpec(memory_space=pltpu.VMEM),
        scratch_shapes=[
            pltpu.VMEM((N_DEV, M, D), jnp.bfloat16),
            pltpu.VMEM((N_DEV, M, D), jnp.float32),
            pltpu.VMEM((N_DEV - 1, M, D), jnp.float32),
            pltpu.VMEM((2, M, D), jnp.float32),
            pltpu.SemaphoreType.DMA((N_DEV - 1,)),
            pltpu.SemaphoreType.DMA((N_DEV - 1,)),
            pltpu.SemaphoreType.DMA((N_DEV - 1,)),
            pltpu.SemaphoreType.DMA((N_DEV - 1,)),
        ],
        compiler_params=pltpu.CompilerParams(collective_id=0),
    )(xb, win0, wout0, win1, wout1, win2, wout2)


# baseline (device time: 411263 ns/iter reference)
import jax
import jax.numpy as jnp
from jax import lax
from jax.experimental import pallas as pl
from jax.experimental.pallas import tpu as pltpu

N_DEV = 16
M = 64
D = 1024
H = 2048
RB = 4


def kernel(x, Win0, Wout0, Win1, Wout1, Win2, Wout2):
    xb = x.astype(jnp.bfloat16)
    win0, wout0, win1, wout1, win2, wout2 = (
        w.astype(jnp.bfloat16) for w in (Win0, Wout0, Win1, Wout1, Win2, Wout2)
    )

    def body(x_ref, win0_r, wout0_r, win1_r, wout1_r, win2_r, wout2_r, out_ref,
             xfull, pacc, rs_recv, rs_send,
             ag_send_sems, ag_recv_sems, rs_send_sems, rs_recv_sems):
        me = lax.axis_index("i")
        right = lax.rem(me + 1, N_DEV)
        left = lax.rem(me + N_DEV - 1, N_DEV)

        barrier = pltpu.get_barrier_semaphore()
        for nbr in (left, right):
            pl.semaphore_signal(barrier, inc=1, device_id=(nbr,),
                                device_id_type=pl.DeviceIdType.MESH)
        pl.semaphore_wait(barrier, 2)

        layers = ((win0_r, wout0_r), (win1_r, wout1_r), (win2_r, wout2_r))
        for l, (win, wout) in enumerate(layers):
            if l == 0:
                xfull[pl.ds(me, 1)] = x_ref[...][None]

            for h in range(1, N_DEV):
                sb = lax.rem(me - h + 1 + N_DEV, N_DEV)
                rb = lax.rem(me - h + N_DEV, N_DEV)
                snd = pltpu.make_async_remote_copy(
                    src_ref=xfull.at[sb], dst_ref=xfull.at[sb],
                    send_sem=ag_send_sems.at[h - 1],
                    recv_sem=ag_recv_sems.at[h - 1],
                    device_id=(right,), device_id_type=pl.DeviceIdType.MESH,
                )
                snd.start()
                snd.wait_send()
                rcv = pltpu.make_async_remote_copy(
                    src_ref=xfull.at[rb], dst_ref=xfull.at[rb],
                    send_sem=ag_send_sems.at[h - 1],
                    recv_sem=ag_recv_sems.at[h - 1],
                    device_id=(left,), device_id_type=pl.DeviceIdType.MESH,
                )
                rcv.wait_recv()

            for g in range(N_DEV // RB):
                xg = xfull[pl.ds(g * RB, RB)].reshape(RB * M, D)
                hg = jnp.maximum(
                    jnp.dot(xg, win[...], preferred_element_type=jnp.float32),
                    0.0,
                ).astype(jnp.bfloat16)
                pg = jnp.dot(hg, wout[...], preferred_element_type=jnp.float32)
                pacc[pl.ds(g * RB, RB)] = pg.reshape(RB, M, D)

            for s in range(1, N_DEV):
                cs = lax.rem(me - s + N_DEV, N_DEV)
                chunk = pacc[pl.ds(cs, 1)][0]
                if s > 1:
                    chunk = chunk + rs_recv[s - 2]
                rs_send[s % 2] = chunk
                hop = pltpu.make_async_remote_copy(
                    src_ref=rs_send.at[s % 2], dst_ref=rs_recv.at[s - 1],
                    send_sem=rs_send_sems.at[s - 1],
                    recv_sem=rs_recv_sems.at[s - 1],
                    device_id=(right,), device_id_type=pl.DeviceIdType.MESH,
                )
                hop.start()
                hop.wait()

            y = pacc[pl.ds(me, 1)][0] + rs_recv[N_DEV - 2]
            if l < 2:
                xfull[pl.ds(me, 1)] = y.astype(jnp.bfloat16)[None]
            else:
                out_ref[...] = y

    return pl.pallas_call(
        body,
        out_shape=jax.ShapeDtypeStruct((M, D), jnp.float32),
        in_specs=[pl.BlockSpec(memory_space=pltpu.VMEM)] * 7,
        out_specs=pl.BlockSpec(memory_space=pltpu.VMEM),
        scratch_shapes=[
            pltpu.VMEM((N_DEV, M, D), jnp.bfloat16),
            pltpu.VMEM((N_DEV, M, D), jnp.float32),
            pltpu.VMEM((N_DEV - 1, M, D), jnp.float32),
            pltpu.VMEM((2, M, D), jnp.float32),
            pltpu.SemaphoreType.DMA((N_DEV - 1,)),
            pltpu.SemaphoreType.DMA((N_DEV - 1,)),
            pltpu.SemaphoreType.DMA((N_DEV - 1,)),
            pltpu.SemaphoreType.DMA((N_DEV - 1,)),
        ],
        compiler_params=pltpu.CompilerParams(collective_id=0),
    )(xb, win0, wout0, win1, wout1, win2, wout2)


# device time: 347659 ns/iter; 1.1829x vs baseline; 1.1829x over previous
import jax
import jax.numpy as jnp
from jax import lax
from jax.experimental import pallas as pl
from jax.experimental.pallas import tpu as pltpu

N_DEV = 16
M = 64
D = 1024
H = 2048


def kernel(x, Win0, Wout0, Win1, Wout1, Win2, Wout2):
    xb = x.astype(jnp.bfloat16)
    win0, wout0, win1, wout1, win2, wout2 = (
        w.astype(jnp.bfloat16) for w in (Win0, Wout0, Win1, Wout1, Win2, Wout2)
    )

    def body(x_ref, win0_r, wout0_r, win1_r, wout1_r, win2_r, wout2_r, out_ref,
             xfull, pacc, rs_recv, rs_send,
             ag_send_sems, ag_recv_sems, rs_send_sems, rs_recv_sems):
        me = lax.axis_index("i")
        right = lax.rem(me + 1, N_DEV)
        left = lax.rem(me + N_DEV - 1, N_DEV)

        barrier = pltpu.get_barrier_semaphore()
        for nbr in (left, right):
            pl.semaphore_signal(barrier, inc=1, device_id=(nbr,),
                                device_id_type=pl.DeviceIdType.MESH)
        pl.semaphore_wait(barrier, 2)

        layers = ((win0_r, wout0_r), (win1_r, wout1_r), (win2_r, wout2_r))
        for l, (win, wout) in enumerate(layers):
            if l == 0:
                xfull[pl.ds(me, 1)] = x_ref[...][None]

            def compute_block(b, win=win, wout=wout):
                xg = xfull[pl.ds(b, 1)][0]
                hg = jnp.maximum(
                    jnp.dot(xg, win[...], preferred_element_type=jnp.float32),
                    0.0,
                ).astype(jnp.bfloat16)
                pacc[pl.ds(b, 1)] = jnp.dot(
                    hg, wout[...], preferred_element_type=jnp.float32
                )[None]

            def ag_rdma(h, nbr):
                sb = lax.rem(me - h + 1 + N_DEV, N_DEV)
                return pltpu.make_async_remote_copy(
                    src_ref=xfull.at[sb], dst_ref=xfull.at[sb],
                    send_sem=ag_send_sems.at[h - 1],
                    recv_sem=ag_recv_sems.at[h - 1],
                    device_id=(nbr,), device_id_type=pl.DeviceIdType.MESH,
                )

            for h in range(1, N_DEV):
                snd = ag_rdma(h, right)
                snd.start()
                snd.wait_send()
                rcv = pltpu.make_async_remote_copy(
                    src_ref=xfull.at[lax.rem(me - h + N_DEV, N_DEV)],
                    dst_ref=xfull.at[lax.rem(me - h + N_DEV, N_DEV)],
                    send_sem=ag_send_sems.at[h - 1],
                    recv_sem=ag_recv_sems.at[h - 1],
                    device_id=(left,), device_id_type=pl.DeviceIdType.MESH,
                )
                rcv.wait_recv()
            for g in range(N_DEV // 4):
                xg = xfull[pl.ds(g * 4, 4)].reshape(4 * M, D)
                hg = jnp.maximum(
                    jnp.dot(xg, win[...], preferred_element_type=jnp.float32),
                    0.0,
                ).astype(jnp.bfloat16)
                pacc[pl.ds(g * 4, 4)] = jnp.dot(
                    hg, wout[...], preferred_element_type=jnp.float32
                ).reshape(4, M, D)

            for s in range(1, N_DEV):
                cs = lax.rem(me - s + N_DEV, N_DEV)
                chunk = pacc[pl.ds(cs, 1)][0]
                if s > 1:
                    chunk = chunk + rs_recv[s - 2].astype(jnp.float32)
                rs_send[s % 2] = chunk.astype(jnp.bfloat16)
                hop = pltpu.make_async_remote_copy(
                    src_ref=rs_send.at[s % 2], dst_ref=rs_recv.at[s - 1],
                    send_sem=rs_send_sems.at[s - 1],
                    recv_sem=rs_recv_sems.at[s - 1],
                    device_id=(right,), device_id_type=pl.DeviceIdType.MESH,
                )
                hop.start()
                hop.wait()

            y = pacc[pl.ds(me, 1)][0] + rs_recv[N_DEV - 2].astype(jnp.float32)
            if l < 2:
                xfull[pl.ds(me, 1)] = y.astype(jnp.bfloat16)[None]
            else:
                out_ref[...] = y

    return pl.pallas_call(
        body,
        out_shape=jax.ShapeDtypeStruct((M, D), jnp.float32),
        in_specs=[pl.BlockSpec(memory_space=pltpu.VMEM)] * 7,
        out_specs=pl.BlockSpec(memory_space=pltpu.VMEM),
        scratch_shapes=[
            pltpu.VMEM((N_DEV, M, D), jnp.bfloat16),
            pltpu.VMEM((N_DEV, M, D), jnp.float32),
            pltpu.VMEM((N_DEV - 1, M, D), jnp.bfloat16),
            pltpu.VMEM((2, M, D), jnp.bfloat16),
            pltpu.SemaphoreType.DMA((N_DEV - 1,)),
            pltpu.SemaphoreType.DMA((N_DEV - 1,)),
            pltpu.SemaphoreType.DMA((N_DEV - 1,)),
            pltpu.SemaphoreType.DMA((N_DEV - 1,)),
        ],
        compiler_params=pltpu.CompilerParams(collective_id=0),
    )(xb, win0, wout0, win1, wout1, win2, wout2)


# device time: 278710 ns/iter; 1.4756x vs baseline; 1.2474x over previous
import jax
import jax.numpy as jnp
from jax import lax
from jax.experimental import pallas as pl
from jax.experimental.pallas import tpu as pltpu

N_DEV = 16
M = 64
D = 1024
H = 2048


def kernel(x, Win0, Wout0, Win1, Wout1, Win2, Wout2):
    xb = x.astype(jnp.bfloat16)
    win0, wout0, win1, wout1, win2, wout2 = (
        w.astype(jnp.bfloat16) for w in (Win0, Wout0, Win1, Wout1, Win2, Wout2)
    )

    def body(x_ref, win0_r, wout0_r, win1_r, wout1_r, win2_r, wout2_r, out_ref,
             xfull, pacc, pb, rs_recv,
             ag_send_sems, ag_recv_sems, rs_send_sems, rs_recv_sems):
        me = lax.axis_index("i")
        right = lax.rem(me + 1, N_DEV)
        left = lax.rem(me + N_DEV - 1, N_DEV)

        barrier = pltpu.get_barrier_semaphore()
        for nbr in (left, right):
            pl.semaphore_signal(barrier, inc=1, device_id=(nbr,),
                                device_id_type=pl.DeviceIdType.MESH)
        pl.semaphore_wait(barrier, 2)

        layers = ((win0_r, wout0_r), (win1_r, wout1_r), (win2_r, wout2_r))
        for l, (win, wout) in enumerate(layers):
            if l == 0:
                xfull[pl.ds(me, 1)] = x_ref[...][None]

            def compute_block(b, win=win, wout=wout):
                xg = xfull[pl.ds(b, 1)][0]
                hg = jnp.maximum(
                    jnp.dot(xg, win[...], preferred_element_type=jnp.float32),
                    0.0,
                ).astype(jnp.bfloat16)
                pacc[pl.ds(b, 1)] = jnp.dot(
                    hg, wout[...], preferred_element_type=jnp.float32
                )[None]

            def ag_rdma(h, nbr):
                sb = lax.rem(me - h + 1 + N_DEV, N_DEV)
                return pltpu.make_async_remote_copy(
                    src_ref=xfull.at[sb], dst_ref=xfull.at[sb],
                    send_sem=ag_send_sems.at[h - 1],
                    recv_sem=ag_recv_sems.at[h - 1],
                    device_id=(nbr,), device_id_type=pl.DeviceIdType.MESH,
                )

            for h in range(1, N_DEV):
                snd = ag_rdma(h, right)
                snd.start()
                snd.wait_send()
                rcv = pltpu.make_async_remote_copy(
                    src_ref=xfull.at[lax.rem(me - h + N_DEV, N_DEV)],
                    dst_ref=xfull.at[lax.rem(me - h + N_DEV, N_DEV)],
                    send_sem=ag_send_sems.at[h - 1],
                    recv_sem=ag_recv_sems.at[h - 1],
                    device_id=(left,), device_id_type=pl.DeviceIdType.MESH,
                )
                rcv.wait_recv()
            for g in range(N_DEV // 4):
                xg = xfull[pl.ds(g * 4, 4)].reshape(4 * M, D)
                hg = jnp.maximum(
                    jnp.dot(xg, win[...], preferred_element_type=jnp.float32),
                    0.0,
                ).astype(jnp.bfloat16)
                pg = jnp.dot(hg, wout[...], preferred_element_type=jnp.float32)
                pacc[pl.ds(g * 4, 4)] = pg.reshape(4, M, D)
                pb[pl.ds(g * 4, 4)] = pg.astype(jnp.bfloat16).reshape(4, M, D)

            sends = []
            for k in range(1, N_DEV):
                dst = lax.rem(me - k + N_DEV, N_DEV)
                snd = pltpu.make_async_remote_copy(
                    src_ref=pb.at[dst], dst_ref=rs_recv.at[k - 1],
                    send_sem=rs_send_sems.at[k - 1],
                    recv_sem=rs_recv_sems.at[k - 1],
                    device_id=(dst,), device_id_type=pl.DeviceIdType.MESH,
                )
                snd.start()
                sends.append(snd)
            for k in range(1, N_DEV):
                rcv = pltpu.make_async_remote_copy(
                    src_ref=rs_recv.at[k - 1], dst_ref=rs_recv.at[k - 1],
                    send_sem=rs_send_sems.at[k - 1],
                    recv_sem=rs_recv_sems.at[k - 1],
                    device_id=(left,), device_id_type=pl.DeviceIdType.MESH,
                )
                rcv.wait_recv()
            for snd in sends:
                snd.wait_send()

            y = pacc[pl.ds(me, 1)][0]
            for k in range(1, N_DEV):
                y = y + rs_recv[k - 1].astype(jnp.float32)
            if l < 2:
                xfull[pl.ds(me, 1)] = y.astype(jnp.bfloat16)[None]
            else:
                out_ref[...] = y

    return pl.pallas_call(
        body,
        out_shape=jax.ShapeDtypeStruct((M, D), jnp.float32),
        in_specs=[pl.BlockSpec(memory_space=pltpu.VMEM)] * 7,
        out_specs=pl.BlockSpec(memory_space=pltpu.VMEM),
        scratch_shapes=[
            pltpu.VMEM((N_DEV, M, D), jnp.bfloat16),
            pltpu.VMEM((N_DEV, M, D), jnp.float32),
            pltpu.VMEM((N_DEV, M, D), jnp.bfloat16),
            pltpu.VMEM((N_DEV - 1, M, D), jnp.bfloat16),
            pltpu.SemaphoreType.DMA((N_DEV - 1,)),
            pltpu.SemaphoreType.DMA((N_DEV - 1,)),
            pltpu.SemaphoreType.DMA((N_DEV - 1,)),
            pltpu.SemaphoreType.DMA((N_DEV - 1,)),
        ],
        compiler_params=pltpu.CompilerParams(collective_id=0),
    )(xb, win0, wout0, win1, wout1, win2, wout2)


# device time: 207234 ns/iter; 1.9845x vs baseline; 1.3449x over previous
import jax
import jax.numpy as jnp
from jax import lax
from jax.experimental import pallas as pl
from jax.experimental.pallas import tpu as pltpu

N_DEV = 16
M = 64
D = 1024
H = 2048


def kernel(x, Win0, Wout0, Win1, Wout1, Win2, Wout2):
    xb = x.astype(jnp.bfloat16)
    win0, wout0, win1, wout1, win2, wout2 = (
        w.astype(jnp.bfloat16) for w in (Win0, Wout0, Win1, Wout1, Win2, Wout2)
    )

    def body(x_ref, win0_r, wout0_r, win1_r, wout1_r, win2_r, wout2_r, out_ref,
             xfull, pacc, pb, rs_recv,
             ag_send_sems, ag_recv_sems, rs_send_sems, rs_recv_sems):
        me = lax.axis_index("i")
        right = lax.rem(me + 1, N_DEV)
        left = lax.rem(me + N_DEV - 1, N_DEV)

        barrier = pltpu.get_barrier_semaphore()
        for k in range(1, N_DEV):
            pl.semaphore_signal(barrier, inc=1,
                                device_id=(lax.rem(me + k, N_DEV),),
                                device_id_type=pl.DeviceIdType.MESH)
        pl.semaphore_wait(barrier, N_DEV - 1)

        layers = ((win0_r, wout0_r), (win1_r, wout1_r), (win2_r, wout2_r))
        for l, (win, wout) in enumerate(layers):
            if l == 0:
                xfull[pl.ds(me, 1)] = x_ref[...][None]

            def compute_block(b, win=win, wout=wout):
                xg = xfull[pl.ds(b, 1)][0]
                hg = jnp.maximum(
                    jnp.dot(xg, win[...], preferred_element_type=jnp.float32),
                    0.0,
                ).astype(jnp.bfloat16)
                pacc[pl.ds(b, 1)] = jnp.dot(
                    hg, wout[...], preferred_element_type=jnp.float32
                )[None]

            ag_sends = []
            for k in range(1, N_DEV):
                snd = pltpu.make_async_remote_copy(
                    src_ref=xfull.at[me], dst_ref=xfull.at[me],
                    send_sem=ag_send_sems.at[k - 1],
                    recv_sem=ag_recv_sems.at[k - 1],
                    device_id=(lax.rem(me + k, N_DEV),),
                    device_id_type=pl.DeviceIdType.MESH,
                )
                snd.start()
                ag_sends.append(snd)
            for k in range(1, N_DEV):
                sb = lax.rem(me - k + N_DEV, N_DEV)
                rcv = pltpu.make_async_remote_copy(
                    src_ref=xfull.at[sb], dst_ref=xfull.at[sb],
                    send_sem=ag_send_sems.at[k - 1],
                    recv_sem=ag_recv_sems.at[k - 1],
                    device_id=(left,), device_id_type=pl.DeviceIdType.MESH,
                )
                rcv.wait_recv()
            for snd in ag_sends:
                snd.wait_send()
            for g in range(N_DEV // 4):
                xg = xfull[pl.ds(g * 4, 4)].reshape(4 * M, D)
                hg = jnp.maximum(
                    jnp.dot(xg, win[...], preferred_element_type=jnp.float32),
                    0.0,
                ).astype(jnp.bfloat16)
                pg = jnp.dot(hg, wout[...], preferred_element_type=jnp.float32)
                pacc[pl.ds(g * 4, 4)] = pg.reshape(4, M, D)
                pb[pl.ds(g * 4, 4)] = pg.astype(jnp.bfloat16).reshape(4, M, D)

            sends = []
            for k in range(1, N_DEV):
                dst = lax.rem(me - k + N_DEV, N_DEV)
                snd = pltpu.make_async_remote_copy(
                    src_ref=pb.at[dst], dst_ref=rs_recv.at[k - 1],
                    send_sem=rs_send_sems.at[k - 1],
                    recv_sem=rs_recv_sems.at[k - 1],
                    device_id=(dst,), device_id_type=pl.DeviceIdType.MESH,
                )
                snd.start()
                sends.append(snd)
            for k in range(1, N_DEV):
                rcv = pltpu.make_async_remote_copy(
                    src_ref=rs_recv.at[k - 1], dst_ref=rs_recv.at[k - 1],
                    send_sem=rs_send_sems.at[k - 1],
                    recv_sem=rs_recv_sems.at[k - 1],
                    device_id=(left,), device_id_type=pl.DeviceIdType.MESH,
                )
                rcv.wait_recv()
            for snd in sends:
                snd.wait_send()

            y = pacc[pl.ds(me, 1)][0]
            for k in range(1, N_DEV):
                y = y + rs_recv[k - 1].astype(jnp.float32)
            if l < 2:
                xfull[pl.ds(me, 1)] = y.astype(jnp.bfloat16)[None]
            else:
                out_ref[...] = y

    return pl.pallas_call(
        body,
        out_shape=jax.ShapeDtypeStruct((M, D), jnp.float32),
        in_specs=[pl.BlockSpec(memory_space=pltpu.VMEM)] * 7,
        out_specs=pl.BlockSpec(memory_space=pltpu.VMEM),
        scratch_shapes=[
            pltpu.VMEM((N_DEV, M, D), jnp.bfloat16),
            pltpu.VMEM((N_DEV, M, D), jnp.float32),
            pltpu.VMEM((N_DEV, M, D), jnp.bfloat16),
            pltpu.VMEM((N_DEV - 1, M, D), jnp.bfloat16),
            pltpu.SemaphoreType.DMA((N_DEV - 1,)),
            pltpu.SemaphoreType.DMA((N_DEV - 1,)),
            pltpu.SemaphoreType.DMA((N_DEV - 1,)),
            pltpu.SemaphoreType.DMA((N_DEV - 1,)),
        ],
        compiler_params=pltpu.CompilerParams(collective_id=0),
    )(xb, win0, wout0, win1, wout1, win2, wout2)
